# baseline (speedup 1.0000x reference)
import functools

import jax
import jax.numpy as jnp
from jax import lax
from jax.experimental import pallas as pl
from jax.experimental.pallas import tpu as pltpu
from jax.experimental.pallas import tpu_sc as plsc

B = 16384
K = 20
D = 64
NC = 2
NS = 16
NW = NC * NS
BPW = B // NW
CB = 32
NCH = BPW // CB
NL = 16


def _body(tw_hbm, cw_hbm, tt_hbm, ct_hbm, out_hbm,
          idx_t, idx_c, t_rows, c_rows, o_buf, sem):
    c = lax.axis_index("c")
    s = lax.axis_index("s")
    wid = s * NC + c
    base = wid * BPW

    def chunk(i, _):
        rb = base + i * CB
        pltpu.sync_copy(tw_hbm.at[pl.ds(rb, CB)], idx_t)
        pltpu.sync_copy(cw_hbm.at[pl.ds(rb * K, CB * K)], idx_c)
        cp_t = pltpu.async_copy(tt_hbm.at[idx_t], t_rows, sem)
        cp_c = pltpu.async_copy(ct_hbm.at[idx_c], c_rows, sem)
        cp_t.wait()
        cp_c.wait()

        lanes = lax.iota(jnp.int32, NL)

        gdn = lax.GatherDimensionNumbers(
            offset_dims=(), collapsed_slice_dims=(0,), start_index_map=(0,))

        def lperm(v, sh):
            return lax.gather(
                v, (lanes ^ sh)[:, None], gdn, (1,),
                mode=lax.GatherScatterMode.PROMISE_IN_BOUNDS)

        def tree_reduce(ps):
            sh = 1
            while len(ps) > 1:
                sel = (lanes & sh) != 0
                ps = [jnp.where(sel, b + lperm(b, sh), a + lperm(a, sh))
                      for a, b in zip(ps[0::2], ps[1::2])]
                sh *= 2
            return ps[0]

        BG = 4
        NACC = BG * K // NL

        def bbody(bq, _):
            b0 = bq * BG
            ps = []
            for bb in range(BG):
                b = b0 + bb
                t0 = t_rows[b, 0:NL]
                t1 = t_rows[b, NL:2 * NL]
                t2 = t_rows[b, 2 * NL:3 * NL]
                t3 = t_rows[b, 3 * NL:4 * NL]
                for k in range(K):
                    row = b * K + k
                    ps.append(t0 * c_rows[row, 0:NL]
                              + t1 * c_rows[row, NL:2 * NL]
                              + t2 * c_rows[row, 2 * NL:3 * NL]
                              + t3 * c_rows[row, 3 * NL:4 * NL])
            for a in range(NACC):
                o_buf[pl.ds(b0 * K + a * NL, NL)] = (
                    tree_reduce(ps[a * NL:(a + 1) * NL]))
            return 0

        lax.fori_loop(0, CB // BG, bbody, 0)
        pltpu.sync_copy(o_buf, out_hbm.at[pl.ds(rb * K, CB * K)])
        return 0

    lax.fori_loop(0, NCH, chunk, 0)


@jax.jit
def _skipgram(target_word, context_word_flat, target_table, context_table):
    mesh = plsc.VectorSubcoreMesh(
        core_axis_name="c", subcore_axis_name="s",
        num_cores=NC, num_subcores=NS)
    f = pl.kernel(
        _body,
        out_type=jax.ShapeDtypeStruct((B * K,), jnp.float32),
        mesh=mesh,
        compiler_params=pltpu.CompilerParams(use_tc_tiling_on_sc=False),
        scratch_types=[
            pltpu.VMEM((CB,), jnp.int32),
            pltpu.VMEM((CB * K,), jnp.int32),
            pltpu.VMEM((CB, D), jnp.float32),
            pltpu.VMEM((CB * K, D), jnp.float32),
            pltpu.VMEM((CB * K,), jnp.float32),
            pltpu.SemaphoreType.DMA,
        ],
    )
    return f(target_word, context_word_flat, target_table, context_table)


def kernel(target_word, context_word, target_table, context_table):
    tw = target_word.astype(jnp.int32)
    cw = context_word.astype(jnp.int32).reshape(B * K)
    out = _skipgram(tw, cw, target_table, context_table)
    return out.reshape(B, K)


# speedup vs baseline: 1.0967x; 1.0967x over previous
import functools

import jax
import jax.numpy as jnp
from jax import lax
from jax.experimental import pallas as pl
from jax.experimental.pallas import tpu as pltpu
from jax.experimental.pallas import tpu_sc as plsc

B = 16384
K = 20
D = 64
NC = 2
NS = 16
NW = NC * NS
BPW = B // NW
CB = 32
NCH = BPW // CB
NL = 16
BG = 4


def _body(tw_hbm, cw_hbm, tt_hbm, ct_hbm, out_hbm,
          idx_t_all, idx_c_all, t_rows, c_rows, o_all,
          semt0, semt1, semc0, semc1):
    c = lax.axis_index("c")
    s = lax.axis_index("s")
    wid = s * NC + c
    base = wid * BPW
    semt = [semt0, semt1]
    semc = [semc0, semc1]

    lanes = lax.iota(jnp.int32, NL)
    gdn = lax.GatherDimensionNumbers(
        offset_dims=(), collapsed_slice_dims=(0,), start_index_map=(0,))

    def lperm(v, sh):
        return lax.gather(
            v, (lanes ^ sh)[:, None], gdn, (1,),
            mode=lax.GatherScatterMode.PROMISE_IN_BOUNDS)

    def combine(a, b, sh):
        sel = (lanes & sh) != 0
        return jnp.where(sel, b + lperm(b, sh), a + lperm(a, sh))

    def gathers(ci, par):
        idx_t_sl = idx_t_all.at[pl.ds(ci * CB, CB)]
        idx_c_sl = idx_c_all.at[pl.ds(ci * CB * K, CB * K)]
        ct = pltpu.make_async_copy(tt_hbm.at[idx_t_sl], t_rows.at[par],
                                   semt[par])
        cc = pltpu.make_async_copy(ct_hbm.at[idx_c_sl], c_rows.at[par],
                                   semc[par])
        return ct, cc

    def issue(ci, par):
        ct, cc = gathers(ci, par)
        ct.start()
        cc.start()

    def compute(ci, par):
        tr = t_rows.at[par]
        cr = c_rows.at[par]

        def bbody(bq, _):
            lb0 = bq * BG
            obase = (ci * CB + lb0) * K
            stack = []
            out_g = 0
            for bb in range(BG):
                lb = lb0 + bb
                t = [tr[lb, pl.ds(q * NL, NL)] for q in range(4)]
                for k in range(K):
                    lrow = lb * K + k
                    p = (t[0] * cr[lrow, 0:NL]
                         + t[1] * cr[lrow, NL:2 * NL]
                         + t[2] * cr[lrow, 2 * NL:3 * NL]
                         + t[3] * cr[lrow, 3 * NL:4 * NL])
                    lvl, node = 0, p
                    while stack and stack[-1][0] == lvl:
                        lv, a = stack.pop()
                        node = combine(a, node, 1 << lv)
                        lvl = lv + 1
                    if lvl == 4:
                        o_all[pl.ds(obase + out_g * NL, NL)] = node
                        out_g += 1
                    else:
                        stack.append((lvl, node))
            return 0

        lax.fori_loop(0, CB // BG, bbody, 0)

    pltpu.sync_copy(tw_hbm.at[pl.ds(base, BPW)], idx_t_all)
    pltpu.sync_copy(cw_hbm.at[pl.ds(base * K, BPW * K)], idx_c_all)

    issue(0, 0)
    issue(1, 1)

    def pair(q, _):
        for par in range(2):
            ci = 2 * q + par
            ct, cc = gathers(ci, par)
            ct.wait()
            cc.wait()
            compute(ci, par)
            nci = ci + 2

            @pl.when(nci < NCH)
            def _():
                issue(nci, par)
        return 0

    lax.fori_loop(0, NCH // 2, pair, 0)

    pltpu.sync_copy(o_all, out_hbm.at[pl.ds(base * K, BPW * K)])


@jax.jit
def _skipgram(target_word, context_word_flat, target_table, context_table):
    mesh = plsc.VectorSubcoreMesh(
        core_axis_name="c", subcore_axis_name="s",
        num_cores=NC, num_subcores=NS)
    f = pl.kernel(
        _body,
        out_type=jax.ShapeDtypeStruct((B * K,), jnp.float32),
        mesh=mesh,
        compiler_params=pltpu.CompilerParams(use_tc_tiling_on_sc=False),
        scratch_types=[
            pltpu.VMEM((BPW,), jnp.int32),
            pltpu.VMEM((BPW * K,), jnp.int32),
            pltpu.VMEM((2, CB, D), jnp.float32),
            pltpu.VMEM((2, CB * K, D), jnp.float32),
            pltpu.VMEM((BPW * K,), jnp.float32),
            pltpu.SemaphoreType.DMA,
            pltpu.SemaphoreType.DMA,
            pltpu.SemaphoreType.DMA,
            pltpu.SemaphoreType.DMA,
        ],
    )
    return f(target_word, context_word_flat, target_table, context_table)


def kernel(target_word, context_word, target_table, context_table):
    tw = target_word.astype(jnp.int32)
    cw = context_word.astype(jnp.int32).reshape(B * K)
    out = _skipgram(tw, cw, target_table, context_table)
    return out.reshape(B, K)
